# baseline (speedup 1.0000x reference)
import functools

import jax
import jax.numpy as jnp
from jax.experimental import pallas as pl
from jax.experimental.pallas import tpu as pltpu

N = 10000
E = 320000
DIN = 128
DH = 64
DOUT = 16
NG = 64
OUT_CAT = 3 * DH + DH * 8


def _mlp_body(h_ref, w1_ref, b1_ref, w2_ref, b2_ref, o1_ref, o2_ref):
    h = h_ref[...]
    z = jnp.maximum(
        jnp.dot(h, w1_ref[...], preferred_element_type=jnp.float32) + b1_ref[...], 0.0
    )
    y = jnp.dot(z, w2_ref[...], preferred_element_type=jnp.float32) + b2_ref[...]
    o1_ref[...] = y
    m = jnp.max(y, axis=1, keepdims=True)
    o2_ref[...] = y - m - jnp.log(jnp.sum(jnp.exp(y - m), axis=1, keepdims=True))


def _mlp_tail(h_cat, W_lin1, b_lin1, W_lin2, b_lin2):
    out_shape = (
        jax.ShapeDtypeStruct((NG, DOUT), jnp.float32),
        jax.ShapeDtypeStruct((NG, DOUT), jnp.float32),
    )
    return pl.pallas_call(
        _mlp_body,
        out_shape=out_shape,
    )(h_cat, W_lin1, b_lin1.reshape(1, -1), W_lin2, b_lin2.reshape(1, -1))


def _gatv2(x, edge_index, Wl, Wr, att, bias, heads, out_ch):
    n = x.shape[0]
    loop = jnp.arange(n)
    src = jnp.concatenate([edge_index[0], loop])
    dst = jnp.concatenate([edge_index[1], loop])
    xl = (x @ Wl).reshape(n, heads, out_ch)
    xr = (x @ Wr).reshape(n, heads, out_ch)
    e = jax.nn.leaky_relu(xl[src] + xr[dst], 0.2)
    alpha = jnp.sum(e * att[None, :, :], axis=-1)
    amax = jax.ops.segment_max(alpha, dst, num_segments=n)
    ex = jnp.exp(alpha - amax[dst])
    denom = jax.ops.segment_sum(ex, dst, num_segments=n)
    a = ex / (denom[dst] + 1e-16)
    out = jax.ops.segment_sum(xl[src] * a[:, :, None], dst, num_segments=n)
    return out.reshape(n, heads * out_ch) + bias


def kernel(x, edge_index, batch, Wl1, Wr1, att1, b1, Wl2, Wr2, att2, b2,
           Wl3, Wr3, att3, b3, W_lin1, b_lin1, W_lin2, b_lin2):
    h1 = jax.nn.elu(_gatv2(x, edge_index, Wl1, Wr1, att1, b1, 8, DH))
    h2 = _gatv2(h1, edge_index, Wl2, Wr2, att2, b2, 1, DH)
    h3 = _gatv2(h2, edge_index, Wl3, Wr3, att3, b3, 1, DH)
    h4 = _gatv2(h3, edge_index, Wl3, Wr3, att3, b3, 1, DH)
    p1 = jax.ops.segment_sum(h1, batch, num_segments=NG)
    p2 = jax.ops.segment_sum(h2, batch, num_segments=NG)
    p3 = jax.ops.segment_sum(h3, batch, num_segments=NG)
    p4 = jax.ops.segment_sum(h4, batch, num_segments=NG)
    h_cat = jnp.concatenate([p1, p2, p3, p4], axis=1)
    return _mlp_tail(h_cat, W_lin1, b_lin1, W_lin2, b_lin2)


# speedup vs baseline: 3.1010x; 3.1010x over previous
import functools

import jax
import jax.numpy as jnp
from jax import lax
from jax.experimental import pallas as pl
from jax.experimental.pallas import tpu as pltpu
from jax.experimental.pallas import tpu_sc as plsc

N = 10000
E = 320000
DIN = 128
DH = 64
DOUT = 16
NG = 64
OUT_CAT = 3 * DH + DH * 8

N_PAD = 10240
SP = N_PAD // 16
K = 80
TW = 128
R = 400
GRID = N // R

f32 = jnp.float32
i32 = jnp.int32


def _pre1_body(x_ref, wl_ref, wr_ref, attf_ref, sel_ref, xl_ref, xr_ref, s_ref):
    x = x_ref[...]
    xl = jnp.dot(x, wl_ref[...], preferred_element_type=f32)
    xr = jnp.dot(x, wr_ref[...], preferred_element_type=f32)
    u = xl + xr
    lr = jnp.maximum(u, 0.2 * u)
    s_ref[...] = jnp.dot(lr * attf_ref[...], sel_ref[...],
                         preferred_element_type=f32)
    xl_ref[...] = xl
    xr_ref[...] = xr


def _pre1(x, Wl1, Wr1, att1f, sel1):
    return pl.pallas_call(
        _pre1_body,
        grid=(GRID,),
        in_specs=[
            pl.BlockSpec((R, DIN), lambda i: (i, 0)),
            pl.BlockSpec((DIN, 8 * DH), lambda i: (0, 0)),
            pl.BlockSpec((DIN, 8 * DH), lambda i: (0, 0)),
            pl.BlockSpec((1, 8 * DH), lambda i: (0, 0)),
            pl.BlockSpec((8 * DH, 8), lambda i: (0, 0)),
        ],
        out_specs=[
            pl.BlockSpec((R, 8 * DH), lambda i: (i, 0)),
            pl.BlockSpec((R, 8 * DH), lambda i: (i, 0)),
            pl.BlockSpec((R, 8), lambda i: (i, 0)),
        ],
        out_shape=[
            jax.ShapeDtypeStruct((N, 8 * DH), f32),
            jax.ShapeDtypeStruct((N, 8 * DH), f32),
            jax.ShapeDtypeStruct((N, 8), f32),
        ],
    )(x, Wl1, Wr1, att1f, sel1)


def _fin1pre2_body(a0_ref, a1_ref, a2_ref, a3_ref,
                   d00_ref, d01_ref, d10_ref, d11_ref,
                   d20_ref, d21_ref, d30_ref, d31_ref,
                   xl1_ref, b1_ref, bat_ref,
                   wl_ref, wr_ref, attf_ref,
                   p_ref, xlr_ref, s2_ref):
    xl1 = xl1_ref[...]
    dens = ((d00_ref, d01_ref), (d10_ref, d11_ref),
            (d20_ref, d21_ref), (d30_ref, d31_ref))
    pieces = []
    for g, ar in enumerate((a0_ref, a1_ref, a2_ref, a3_ref)):
        a = ar[...]
        xg = xl1[:, 128 * g:128 * (g + 1)]
        d0 = jnp.sum(dens[g][0][...].reshape(16, R), axis=0)[:, None] + 1.0
        d1 = jnp.sum(dens[g][1][...].reshape(16, R), axis=0)[:, None] + 1.0
        pieces.append((a[:, 0:64] + xg[:, 0:64]) / d0)
        pieces.append((a[:, 64:128] + xg[:, 64:128]) / d1)
    h = jnp.concatenate(pieces, axis=1) + b1_ref[...]
    h = jnp.where(h > 0, h, jnp.exp(jnp.minimum(h, 0.0)) - 1.0)
    bv = bat_ref[...].reshape(R)
    oh = (bv[:, None] == lax.broadcasted_iota(i32, (R, NG), 1)).astype(f32)

    @pl.when(pl.program_id(0) == 0)
    def _():
        p_ref[...] = jnp.zeros_like(p_ref)

    p_ref[...] += lax.dot_general(oh, h, (((0,), (0,)), ((), ())),
                                  preferred_element_type=f32)
    xl2 = jnp.dot(h, wl_ref[...], preferred_element_type=f32)
    xr2 = jnp.dot(h, wr_ref[...], preferred_element_type=f32)
    u = xl2 + xr2
    lr = jnp.maximum(u, 0.2 * u)
    s2_ref[...] = jnp.sum(lr * attf_ref[...], axis=1, keepdims=True)
    xlr_ref[...] = jnp.concatenate([xl2, xr2], axis=1)


def _fin1pre2(accs, den_pairs, xl1, b1, bat3, Wl2, Wr2, att2f):
    dens_flat = [d for pair in den_pairs for d in pair]
    return pl.pallas_call(
        _fin1pre2_body,
        grid=(GRID,),
        in_specs=(
            [pl.BlockSpec((R, TW), lambda i: (i, 0))] * 4
            + [pl.BlockSpec((1, 16, R), lambda i: (i, 0, 0))] * 8
            + [
                pl.BlockSpec((R, 8 * DH), lambda i: (i, 0)),
                pl.BlockSpec((1, 8 * DH), lambda i: (0, 0)),
                pl.BlockSpec((1, 1, R), lambda i: (i, 0, 0)),
                pl.BlockSpec((8 * DH, DH), lambda i: (0, 0)),
                pl.BlockSpec((8 * DH, DH), lambda i: (0, 0)),
                pl.BlockSpec((1, DH), lambda i: (0, 0)),
            ]
        ),
        out_specs=[
            pl.BlockSpec((NG, 8 * DH), lambda i: (0, 0)),
            pl.BlockSpec((R, TW), lambda i: (i, 0)),
            pl.BlockSpec((R, 1), lambda i: (i, 0)),
        ],
        out_shape=[
            jax.ShapeDtypeStruct((NG, 8 * DH), f32),
            jax.ShapeDtypeStruct((N, TW), f32),
            jax.ShapeDtypeStruct((N, 1), f32),
        ],
    )(*accs, *dens_flat, xl1, b1, bat3, Wl2, Wr2, att2f)


def _finpre_body(aa_ref, ab_ref, da_ref, xlr_ref, b_ref, bat_ref,
                 wl_ref, wr_ref, attf_ref, p_ref, xlrn_ref, sn_ref):
    d = jnp.sum(da_ref[...].reshape(32, R), axis=0)[:, None] + 1.0
    xl = xlr_ref[...][:, 0:DH]
    h = (aa_ref[...][:, 0:DH] + ab_ref[...][:, 0:DH] + xl) / d + b_ref[...]
    bv = bat_ref[...].reshape(R)
    oh = (bv[:, None] == lax.broadcasted_iota(i32, (R, NG), 1)).astype(f32)

    @pl.when(pl.program_id(0) == 0)
    def _():
        p_ref[...] = jnp.zeros_like(p_ref)

    p_ref[...] += lax.dot_general(oh, h, (((0,), (0,)), ((), ())),
                                  preferred_element_type=f32)
    xln = jnp.dot(h, wl_ref[...], preferred_element_type=f32)
    xrn = jnp.dot(h, wr_ref[...], preferred_element_type=f32)
    u = xln + xrn
    lr = jnp.maximum(u, 0.2 * u)
    sn_ref[...] = jnp.sum(lr * attf_ref[...], axis=1, keepdims=True)
    xlrn_ref[...] = jnp.concatenate([xln, xrn], axis=1)


def _finpre(aa, ab, da, xlr, b, bat3, Wln, Wrn, attnf):
    return pl.pallas_call(
        _finpre_body,
        grid=(GRID,),
        in_specs=[
            pl.BlockSpec((R, TW), lambda i: (i, 0)),
            pl.BlockSpec((R, TW), lambda i: (i, 0)),
            pl.BlockSpec((1, 32, R), lambda i: (i, 0, 0)),
            pl.BlockSpec((R, TW), lambda i: (i, 0)),
            pl.BlockSpec((1, DH), lambda i: (0, 0)),
            pl.BlockSpec((1, 1, R), lambda i: (i, 0, 0)),
            pl.BlockSpec((DH, DH), lambda i: (0, 0)),
            pl.BlockSpec((DH, DH), lambda i: (0, 0)),
            pl.BlockSpec((1, DH), lambda i: (0, 0)),
        ],
        out_specs=[
            pl.BlockSpec((NG, DH), lambda i: (0, 0)),
            pl.BlockSpec((R, TW), lambda i: (i, 0)),
            pl.BlockSpec((R, 1), lambda i: (i, 0)),
        ],
        out_shape=[
            jax.ShapeDtypeStruct((NG, DH), f32),
            jax.ShapeDtypeStruct((N, TW), f32),
            jax.ShapeDtypeStruct((N, 1), f32),
        ],
    )(aa, ab, da, xlr, b, bat3, Wln, Wrn, attnf)


def _fin4_body(aa_ref, ab_ref, da_ref, xlr_ref, b_ref, bat_ref, p_ref):
    d = jnp.sum(da_ref[...].reshape(32, R), axis=0)[:, None] + 1.0
    xl = xlr_ref[...][:, 0:DH]
    h = (aa_ref[...][:, 0:DH] + ab_ref[...][:, 0:DH] + xl) / d + b_ref[...]
    bv = bat_ref[...].reshape(R)
    oh = (bv[:, None] == lax.broadcasted_iota(i32, (R, NG), 1)).astype(f32)

    @pl.when(pl.program_id(0) == 0)
    def _():
        p_ref[...] = jnp.zeros_like(p_ref)

    p_ref[...] += lax.dot_general(oh, h, (((0,), (0,)), ((), ())),
                                  preferred_element_type=f32)


def _fin4(aa, ab, da, xlr, b, bat3):
    return pl.pallas_call(
        _fin4_body,
        grid=(GRID,),
        in_specs=[
            pl.BlockSpec((R, TW), lambda i: (i, 0)),
            pl.BlockSpec((R, TW), lambda i: (i, 0)),
            pl.BlockSpec((1, 32, R), lambda i: (i, 0, 0)),
            pl.BlockSpec((R, TW), lambda i: (i, 0)),
            pl.BlockSpec((1, DH), lambda i: (0, 0)),
            pl.BlockSpec((1, 1, R), lambda i: (i, 0, 0)),
        ],
        out_specs=[pl.BlockSpec((NG, DH), lambda i: (0, 0))],
        out_shape=[jax.ShapeDtypeStruct((NG, DH), f32)],
    )(aa, ab, da, xlr, b, bat3)


def _mlp_body(h_ref, w1_ref, b1_ref, w2_ref, b2_ref, o1_ref, o2_ref):
    h = h_ref[...]
    z = jnp.maximum(
        jnp.dot(h, w1_ref[...], preferred_element_type=f32) + b1_ref[...], 0.0)
    y = jnp.dot(z, w2_ref[...], preferred_element_type=f32) + b2_ref[...]
    o1_ref[...] = y
    m = jnp.max(y, axis=1, keepdims=True)
    o2_ref[...] = y - m - jnp.log(jnp.sum(jnp.exp(y - m), axis=1, keepdims=True))


def _mlp_tail(h_cat, W_lin1, b_lin1, W_lin2, b_lin2):
    return pl.pallas_call(
        _mlp_body,
        out_shape=(
            jax.ShapeDtypeStruct((NG, DOUT), f32),
            jax.ShapeDtypeStruct((NG, DOUT), f32),
        ),
    )(h_cat, W_lin1, b_lin1.reshape(1, -1), W_lin2, b_lin2.reshape(1, -1))


def _sc_edge_body(Deff, H2, xr_col0, ept, split_edges, per_core_tables,
                  xltab, xrtab, s_hbm, att_hbm, src_hbm, dst_hbm,
                  zeros_hbm, zerosd_hbm,
                  out_data, out_den0, out_den1,
                  att_v, srcb, dstb, srcadj, dstadj, sidx0, sidx1,
                  sbuf0, sbuf1, xlb, xrb, scatb, den0, den1, acc):
    inplace = per_core_tables
    core = lax.axis_index("c")
    sub = lax.axis_index("s")
    iota16 = lax.iota(i32, 16)

    aroff = core * 16
    soff = core * N * H2 if per_core_tables else 0
    pltpu.sync_copy(att_hbm.at[pl.ds(aroff, 16)], att_v)
    pltpu.sync_copy(zeros_hbm.at[pl.ds(sub * SP, SP)], acc.at[pl.ds(sub * SP, SP)])
    if not inplace:
        pltpu.sync_copy(zeros_hbm.at[pl.ds(0, K)], scatb)
    pltpu.sync_copy(zerosd_hbm.at[pl.ds(0, N)], den0)
    if H2 == 2:
        pltpu.sync_copy(zerosd_hbm.at[pl.ds(0, N)], den1)
    plsc.subcore_barrier()

    base = (core * 16 + sub) * ept if split_edges else sub * ept
    nchunks = ept // K

    def chunk_body(t, carry):
        e0 = base + t * K
        pltpu.sync_copy(src_hbm.at[pl.ds(e0, K)], srcb)
        pltpu.sync_copy(dst_hbm.at[pl.ds(e0, K)], dstb)
        for j in range(K // 16):
            sl = pl.ds(j * 16, 16)
            dvj = dstb[sl]
            sidx0[sl] = dvj * H2 + soff
            if H2 == 2:
                sidx1[sl] = dvj * H2 + (soff + 1)
            if per_core_tables:
                off = core * N
                srcadj[sl] = srcb[sl] + off
                dstadj[sl] = dvj + off
        if per_core_tables:
            src_idx, dst_idx = srcadj, dstadj
        else:
            src_idx, dst_idx = srcb, dstb
        pltpu.sync_copy(s_hbm.at[sidx0], sbuf0)
        if H2 == 2:
            pltpu.sync_copy(s_hbm.at[sidx1], sbuf1)
        pltpu.sync_copy(xltab.at[src_idx], xlb)
        pltpu.sync_copy(xrtab.at[dst_idx], xrb)

        def grp(g, c2):
            rows = iota16 + g * 16
            dv = dstb[pl.ds(g * 16, 16)]
            zs = []
            svs = [sbuf0, sbuf1]
            for h in range(H2):
                def ablk(b, acc, h=h):
                    for ci in range(16):
                        colv = jnp.full((16,), ci, i32) + (b * 16 + h * DH)
                        xlc = plsc.load_gather(xlb, [rows, colv])
                        xrc = plsc.load_gather(xrb, [rows, colv + xr_col0])
                        u = xlc + xrc
                        lr = jnp.maximum(u, 0.2 * u)
                        attc = plsc.load_gather(att_v, [iota16, colv])
                        acc = acc + lr * attc
                    return acc
                a = lax.fori_loop(0, DH // 16, ablk, jnp.zeros((16,), f32))
                sv = svs[h][pl.ds(g * 16, 16)]
                zs.append(jnp.exp(jnp.minimum(a - sv, 60.0)))
            dst_buf = xlb if inplace else scatb

            def sblk(b, c3):
                zz = jnp.where(b * 16 >= DH, zs[-1], zs[0])
                for ci in range(16):
                    colv = jnp.full((16,), ci, i32) + b * 16
                    xlc = plsc.load_gather(xlb, [rows, colv])
                    plsc.store_scatter(dst_buf, [rows, colv], xlc * zz)
                return c3

            lax.fori_loop(0, Deff // 16, sblk, 0)
            for j in range(16):
                m = iota16 == j
                plsc.addupdate_scatter(den0, [dv], zs[0], mask=m)
                if H2 == 2:
                    plsc.addupdate_scatter(den1, [dv], zs[1], mask=m)
            return c2

        lax.fori_loop(0, K // 16, grp, 0)
        pltpu.sync_copy(xlb if inplace else scatb, acc.at[dstb], add=True)
        return carry

    lax.fori_loop(0, nchunks, chunk_body, 0)
    plsc.subcore_barrier()
    woff = core * N_PAD + sub * SP
    pltpu.sync_copy(acc.at[pl.ds(sub * SP, SP)], out_data.at[pl.ds(woff, SP)])
    wid = core * 16 + sub
    pltpu.sync_copy(den0, out_den0.at[wid])
    if H2 == 2:
        pltpu.sync_copy(den1, out_den1.at[wid])


def _sc_edge(xltab, xrtab, s, attf, src, dst, zeros, zerosd, Deff, H2, xr_col0,
             ept, split_edges, per_core_tables):
    inplace = per_core_tables
    mesh = plsc.VectorSubcoreMesh(core_axis_name="c", subcore_axis_name="s",
                                  num_cores=2, num_subcores=16)
    body = functools.partial(_sc_edge_body, Deff, H2, xr_col0, ept,
                             split_edges, per_core_tables)
    kern = pl.kernel(
        body,
        out_type=[
            jax.ShapeDtypeStruct((2 * N_PAD, TW), f32),
            jax.ShapeDtypeStruct((32, N), f32),
            jax.ShapeDtypeStruct((32, N), f32),
        ],
        mesh=mesh,
        compiler_params=pltpu.CompilerParams(needs_layout_passes=False),
        scratch_types=[
            pltpu.VMEM((16, TW), f32),
            pltpu.VMEM((K,), i32),
            pltpu.VMEM((K,), i32),
            pltpu.VMEM((K,), i32),
            pltpu.VMEM((K,), i32),
            pltpu.VMEM((K,), i32),
            pltpu.VMEM((K,) if H2 == 2 else (16,), i32),
            pltpu.VMEM((K,), f32),
            pltpu.VMEM((K,) if H2 == 2 else (16,), f32),
            pltpu.VMEM((K, TW), f32),
            pltpu.VMEM((K, TW), f32),
            pltpu.VMEM((8, TW) if inplace else (K, TW), f32),
            pltpu.VMEM((N,), f32),
            pltpu.VMEM((N,) if H2 == 2 else (16,), f32),
            pltpu.VMEM_SHARED((N_PAD, TW), f32),
        ],
    )
    return kern(xltab, xrtab, s, attf, src, dst, zeros, zerosd)


def kernel(x, edge_index, batch, Wl1, Wr1, att1, b1, Wl2, Wr2, att2, b2,
           Wl3, Wr3, att3, b3, W_lin1, b_lin1, W_lin2, b_lin2):
    src = edge_index[0].astype(i32)
    dst = edge_index[1].astype(i32)
    bat3 = batch.astype(i32).reshape(GRID, 1, R)

    att1f = att1.reshape(1, 8 * DH)
    att2f = att2.reshape(1, DH)
    att3f = att3.reshape(1, DH)
    sel1 = (jnp.arange(8 * DH)[:, None] // DH
            == jnp.arange(8)[None, :]).astype(f32)
    zeros_tw = jnp.zeros((N_PAD, TW), f32)
    zerosd = jnp.zeros((N,), f32)

    xl1, xr1, s1 = _pre1(x, Wl1, Wr1, att1f, sel1)

    accs, den_pairs = [], []
    for c in range(2):
        lo = 256 * c
        xl_pair = jnp.concatenate([xl1[:, lo:lo + 128], xl1[:, lo + 128:lo + 256]],
                                  axis=0)
        xr_pair = jnp.concatenate([xr1[:, lo:lo + 128], xr1[:, lo + 128:lo + 256]],
                                  axis=0)
        s_pair = jnp.concatenate([s1[:, 4 * c:4 * c + 2], s1[:, 4 * c + 2:4 * c + 4]],
                                 axis=0)
        att_pair = jnp.concatenate([jnp.tile(att1f[:, lo:lo + 128], (16, 1)),
                                    jnp.tile(att1f[:, lo + 128:lo + 256], (16, 1))],
                                   axis=0)
        data, dn0, dn1 = _sc_edge(xl_pair, xr_pair, s_pair.reshape(-1), att_pair, src, dst,
                                  zeros_tw, zerosd, Deff=128, H2=2, xr_col0=0,
                                  ept=E // 16, split_edges=False,
                                  per_core_tables=True)
        accs.append(data[0:N])
        accs.append(data[N_PAD:N_PAD + N])
        den_pairs.append((dn0[0:16].reshape(16, GRID, R).transpose(1, 0, 2),
                          dn1[0:16].reshape(16, GRID, R).transpose(1, 0, 2)))
        den_pairs.append((dn0[16:32].reshape(16, GRID, R).transpose(1, 0, 2),
                          dn1[16:32].reshape(16, GRID, R).transpose(1, 0, 2)))

    p1, xlr2, s2 = _fin1pre2(accs, den_pairs, xl1, b1.reshape(1, -1), bat3,
                             Wl2, Wr2, att2f)

    data2, dn2, _ = _sc_edge(xlr2, xlr2, s2.reshape(-1), jnp.tile(jnp.pad(att2f, ((0, 0), (0, TW - DH))), (32, 1)), src, dst,
                             zeros_tw, zerosd, Deff=DH, H2=1, xr_col0=DH,
                             ept=E // 32, split_edges=True,
                             per_core_tables=False)
    p2, xlr3, s3 = _finpre(data2[0:N], data2[N_PAD:N_PAD + N], dn2.reshape(32, GRID, R).transpose(1, 0, 2), xlr2,
                           b2.reshape(1, -1), bat3, Wl3, Wr3, att3f)

    data3, dn3, _ = _sc_edge(xlr3, xlr3, s3.reshape(-1), jnp.tile(jnp.pad(att3f, ((0, 0), (0, TW - DH))), (32, 1)), src, dst,
                             zeros_tw, zerosd, Deff=DH, H2=1, xr_col0=DH,
                             ept=E // 32, split_edges=True,
                             per_core_tables=False)
    p3, xlr4, s4 = _finpre(data3[0:N], data3[N_PAD:N_PAD + N], dn3.reshape(32, GRID, R).transpose(1, 0, 2), xlr3,
                           b3.reshape(1, -1), bat3, Wl3, Wr3, att3f)

    data4, dn4, _ = _sc_edge(xlr4, xlr4, s4.reshape(-1), jnp.tile(jnp.pad(att3f, ((0, 0), (0, TW - DH))), (32, 1)), src, dst,
                             zeros_tw, zerosd, Deff=DH, H2=1, xr_col0=DH,
                             ept=E // 32, split_edges=True,
                             per_core_tables=False)
    p4 = _fin4(data4[0:N], data4[N_PAD:N_PAD + N], dn4.reshape(32, GRID, R).transpose(1, 0, 2), xlr4,
               b3.reshape(1, -1), bat3)[0]

    h_cat = jnp.concatenate([p1, p2, p3, p4], axis=1)
    return _mlp_tail(h_cat, W_lin1, b_lin1, W_lin2, b_lin2)


# speedup vs baseline: 3.2749x; 1.0561x over previous
import functools

import jax
import jax.numpy as jnp
from jax import lax
from jax.experimental import pallas as pl
from jax.experimental.pallas import tpu as pltpu
from jax.experimental.pallas import tpu_sc as plsc

N = 10000
E = 320000
DIN = 128
DH = 64
DOUT = 16
NG = 64
OUT_CAT = 3 * DH + DH * 8

N_PAD = 10240
SP = N_PAD // 16
K = 80
SUP = 400
TW = 128
R = 400
GRID = N // R

f32 = jnp.float32
i32 = jnp.int32


def _pre1_body(x_ref, wl_ref, wr_ref, attf_ref, sel_ref, xl_ref, xr_ref, s_ref):
    x = x_ref[...]
    xl = jnp.dot(x, wl_ref[...], preferred_element_type=f32)
    xr = jnp.dot(x, wr_ref[...], preferred_element_type=f32)
    u = xl + xr
    lr = jnp.maximum(u, 0.2 * u)
    s_ref[...] = jnp.dot(lr * attf_ref[...], sel_ref[...],
                         preferred_element_type=f32)
    xl_ref[...] = xl
    xr_ref[...] = xr


def _pre1(x, Wl1, Wr1, att1f, sel1):
    return pl.pallas_call(
        _pre1_body,
        grid=(GRID,),
        in_specs=[
            pl.BlockSpec((R, DIN), lambda i: (i, 0)),
            pl.BlockSpec((DIN, 8 * DH), lambda i: (0, 0)),
            pl.BlockSpec((DIN, 8 * DH), lambda i: (0, 0)),
            pl.BlockSpec((1, 8 * DH), lambda i: (0, 0)),
            pl.BlockSpec((8 * DH, 8), lambda i: (0, 0)),
        ],
        out_specs=[
            pl.BlockSpec((R, 8 * DH), lambda i: (i, 0)),
            pl.BlockSpec((R, 8 * DH), lambda i: (i, 0)),
            pl.BlockSpec((R, 8), lambda i: (i, 0)),
        ],
        out_shape=[
            jax.ShapeDtypeStruct((N, 8 * DH), f32),
            jax.ShapeDtypeStruct((N, 8 * DH), f32),
            jax.ShapeDtypeStruct((N, 8), f32),
        ],
    )(x, Wl1, Wr1, att1f, sel1)


def _fin1pre2_body(a0_ref, a1_ref, a2_ref, a3_ref,
                   d00_ref, d01_ref, d10_ref, d11_ref,
                   d20_ref, d21_ref, d30_ref, d31_ref,
                   xl1_ref, b1_ref, bat_ref,
                   wl_ref, wr_ref, attf_ref,
                   p_ref, xlr_ref, s2_ref):
    xl1 = xl1_ref[...]
    dens = ((d00_ref, d01_ref), (d10_ref, d11_ref),
            (d20_ref, d21_ref), (d30_ref, d31_ref))
    pieces = []
    for g, ar in enumerate((a0_ref, a1_ref, a2_ref, a3_ref)):
        a = ar[...]
        xg = xl1[:, 128 * g:128 * (g + 1)]
        d0 = jnp.sum(dens[g][0][...].reshape(16, R), axis=0)[:, None] + 1.0
        d1 = jnp.sum(dens[g][1][...].reshape(16, R), axis=0)[:, None] + 1.0
        pieces.append((a[:, 0:64] + xg[:, 0:64]) / d0)
        pieces.append((a[:, 64:128] + xg[:, 64:128]) / d1)
    h = jnp.concatenate(pieces, axis=1) + b1_ref[...]
    h = jnp.where(h > 0, h, jnp.exp(jnp.minimum(h, 0.0)) - 1.0)
    bv = bat_ref[...].reshape(R)
    oh = (bv[:, None] == lax.broadcasted_iota(i32, (R, NG), 1)).astype(f32)

    @pl.when(pl.program_id(0) == 0)
    def _():
        p_ref[...] = jnp.zeros_like(p_ref)

    p_ref[...] += lax.dot_general(oh, h, (((0,), (0,)), ((), ())),
                                  preferred_element_type=f32)
    xl2 = jnp.dot(h, wl_ref[...], preferred_element_type=f32)
    xr2 = jnp.dot(h, wr_ref[...], preferred_element_type=f32)
    u = xl2 + xr2
    lr = jnp.maximum(u, 0.2 * u)
    s2_ref[...] = jnp.sum(lr * attf_ref[...], axis=1, keepdims=True)
    xlr_ref[...] = jnp.concatenate([xl2, xr2], axis=1)


def _fin1pre2(accs, den_pairs, xl1, b1, bat3, Wl2, Wr2, att2f):
    dens_flat = [d for pair in den_pairs for d in pair]
    return pl.pallas_call(
        _fin1pre2_body,
        grid=(GRID,),
        in_specs=(
            [pl.BlockSpec((R, TW), lambda i: (i, 0))] * 4
            + [pl.BlockSpec((1, 16, R), lambda i: (i, 0, 0))] * 8
            + [
                pl.BlockSpec((R, 8 * DH), lambda i: (i, 0)),
                pl.BlockSpec((1, 8 * DH), lambda i: (0, 0)),
                pl.BlockSpec((1, 1, R), lambda i: (i, 0, 0)),
                pl.BlockSpec((8 * DH, DH), lambda i: (0, 0)),
                pl.BlockSpec((8 * DH, DH), lambda i: (0, 0)),
                pl.BlockSpec((1, DH), lambda i: (0, 0)),
            ]
        ),
        out_specs=[
            pl.BlockSpec((NG, 8 * DH), lambda i: (0, 0)),
            pl.BlockSpec((R, TW), lambda i: (i, 0)),
            pl.BlockSpec((R, 1), lambda i: (i, 0)),
        ],
        out_shape=[
            jax.ShapeDtypeStruct((NG, 8 * DH), f32),
            jax.ShapeDtypeStruct((N, TW), f32),
            jax.ShapeDtypeStruct((N, 1), f32),
        ],
    )(*accs, *dens_flat, xl1, b1, bat3, Wl2, Wr2, att2f)


def _finpre_body(aa_ref, ab_ref, da_ref, xlr_ref, b_ref, bat_ref,
                 wl_ref, wr_ref, attf_ref, p_ref, xlrn_ref, sn_ref):
    d = jnp.sum(da_ref[...].reshape(32, R), axis=0)[:, None] + 1.0
    xl = xlr_ref[...][:, 0:DH]
    h = (aa_ref[...][:, 0:DH] + ab_ref[...][:, 0:DH] + xl) / d + b_ref[...]
    bv = bat_ref[...].reshape(R)
    oh = (bv[:, None] == lax.broadcasted_iota(i32, (R, NG), 1)).astype(f32)

    @pl.when(pl.program_id(0) == 0)
    def _():
        p_ref[...] = jnp.zeros_like(p_ref)

    p_ref[...] += lax.dot_general(oh, h, (((0,), (0,)), ((), ())),
                                  preferred_element_type=f32)
    xln = jnp.dot(h, wl_ref[...], preferred_element_type=f32)
    xrn = jnp.dot(h, wr_ref[...], preferred_element_type=f32)
    u = xln + xrn
    lr = jnp.maximum(u, 0.2 * u)
    sn_ref[...] = jnp.sum(lr * attf_ref[...], axis=1, keepdims=True)
    xlrn_ref[...] = jnp.concatenate([xln, xrn], axis=1)


def _finpre(aa, ab, da, xlr, b, bat3, Wln, Wrn, attnf):
    return pl.pallas_call(
        _finpre_body,
        grid=(GRID,),
        in_specs=[
            pl.BlockSpec((R, TW), lambda i: (i, 0)),
            pl.BlockSpec((R, TW), lambda i: (i, 0)),
            pl.BlockSpec((1, 32, R), lambda i: (i, 0, 0)),
            pl.BlockSpec((R, TW), lambda i: (i, 0)),
            pl.BlockSpec((1, DH), lambda i: (0, 0)),
            pl.BlockSpec((1, 1, R), lambda i: (i, 0, 0)),
            pl.BlockSpec((DH, DH), lambda i: (0, 0)),
            pl.BlockSpec((DH, DH), lambda i: (0, 0)),
            pl.BlockSpec((1, DH), lambda i: (0, 0)),
        ],
        out_specs=[
            pl.BlockSpec((NG, DH), lambda i: (0, 0)),
            pl.BlockSpec((R, TW), lambda i: (i, 0)),
            pl.BlockSpec((R, 1), lambda i: (i, 0)),
        ],
        out_shape=[
            jax.ShapeDtypeStruct((NG, DH), f32),
            jax.ShapeDtypeStruct((N, TW), f32),
            jax.ShapeDtypeStruct((N, 1), f32),
        ],
    )(aa, ab, da, xlr, b, bat3, Wln, Wrn, attnf)


def _fin4_body(aa_ref, ab_ref, da_ref, xlr_ref, b_ref, bat_ref, p_ref):
    d = jnp.sum(da_ref[...].reshape(32, R), axis=0)[:, None] + 1.0
    xl = xlr_ref[...][:, 0:DH]
    h = (aa_ref[...][:, 0:DH] + ab_ref[...][:, 0:DH] + xl) / d + b_ref[...]
    bv = bat_ref[...].reshape(R)
    oh = (bv[:, None] == lax.broadcasted_iota(i32, (R, NG), 1)).astype(f32)

    @pl.when(pl.program_id(0) == 0)
    def _():
        p_ref[...] = jnp.zeros_like(p_ref)

    p_ref[...] += lax.dot_general(oh, h, (((0,), (0,)), ((), ())),
                                  preferred_element_type=f32)


def _fin4(aa, ab, da, xlr, b, bat3):
    return pl.pallas_call(
        _fin4_body,
        grid=(GRID,),
        in_specs=[
            pl.BlockSpec((R, TW), lambda i: (i, 0)),
            pl.BlockSpec((R, TW), lambda i: (i, 0)),
            pl.BlockSpec((1, 32, R), lambda i: (i, 0, 0)),
            pl.BlockSpec((R, TW), lambda i: (i, 0)),
            pl.BlockSpec((1, DH), lambda i: (0, 0)),
            pl.BlockSpec((1, 1, R), lambda i: (i, 0, 0)),
        ],
        out_specs=[pl.BlockSpec((NG, DH), lambda i: (0, 0))],
        out_shape=[jax.ShapeDtypeStruct((NG, DH), f32)],
    )(aa, ab, da, xlr, b, bat3)


def _mlp_body(h_ref, w1_ref, b1_ref, w2_ref, b2_ref, o1_ref, o2_ref):
    h = h_ref[...]
    z = jnp.maximum(
        jnp.dot(h, w1_ref[...], preferred_element_type=f32) + b1_ref[...], 0.0)
    y = jnp.dot(z, w2_ref[...], preferred_element_type=f32) + b2_ref[...]
    o1_ref[...] = y
    m = jnp.max(y, axis=1, keepdims=True)
    o2_ref[...] = y - m - jnp.log(jnp.sum(jnp.exp(y - m), axis=1, keepdims=True))


def _mlp_tail(h_cat, W_lin1, b_lin1, W_lin2, b_lin2):
    return pl.pallas_call(
        _mlp_body,
        out_shape=(
            jax.ShapeDtypeStruct((NG, DOUT), f32),
            jax.ShapeDtypeStruct((NG, DOUT), f32),
        ),
    )(h_cat, W_lin1, b_lin1.reshape(1, -1), W_lin2, b_lin2.reshape(1, -1))


def _sc_edge_body(Deff, H2, xr_col0, ept, split_edges, per_core_tables,
                  xltab, xrtab, s_hbm, att_hbm, src_hbm, dst_hbm,
                  zeros_hbm, zerosd_hbm,
                  out_data, out_den0, out_den1,
                  att_v, srcb, dstadj, srcS, dstS, sidx0, sidx1,
                  sbuf0, sbuf1, xlb, xrb, den0, den1, dstb, acc):
    core = lax.axis_index("c")
    sub = lax.axis_index("s")
    iota16 = lax.iota(i32, 16)

    aroff = core * 16
    soff = core * N * H2 if per_core_tables else 0
    pltpu.sync_copy(att_hbm.at[pl.ds(aroff, 16)], att_v)
    pltpu.sync_copy(zeros_hbm.at[pl.ds(sub * SP, SP)], acc.at[pl.ds(sub * SP, SP)])
    pltpu.sync_copy(zerosd_hbm.at[pl.ds(0, N)], den0)
    if H2 == 2:
        pltpu.sync_copy(zerosd_hbm.at[pl.ds(0, N)], den1)
    plsc.subcore_barrier()

    base = (core * 16 + sub) * ept if split_edges else sub * ept
    coff = core * N if per_core_tables else 0

    def sup_body(ts, carry):
        e0 = base + ts * SUP
        pltpu.sync_copy(src_hbm.at[pl.ds(e0, SUP)], srcS)
        pltpu.sync_copy(dst_hbm.at[pl.ds(e0, SUP)], dstS)
        for j in range(SUP // 16):
            sl = pl.ds(j * 16, 16)
            dvj = dstS[sl]
            sidx0[sl] = dvj * H2 + soff
            if H2 == 2:
                sidx1[sl] = dvj * H2 + (soff + 1)
        pltpu.sync_copy(s_hbm.at[sidx0], sbuf0)
        if H2 == 2:
            pltpu.sync_copy(s_hbm.at[sidx1], sbuf1)

        def chunk_body(tc, c1):
            for j in range(K // 16):
                sls = pl.ds(j * 16, 16)
                slb = pl.ds(tc * K + j * 16, 16)
                srcb[sls] = srcS[slb] + coff
                dstb[sls] = dstS[slb]
                dstadj[sls] = dstS[slb] + coff
            pltpu.sync_copy(xltab.at[srcb], xlb)
            pltpu.sync_copy(xrtab.at[dstadj], xrb)

            def grp(g, c2):
                rows = iota16 + g * 16
                dv = dstb[pl.ds(g * 16, 16)]
                zs = []
                svs = [sbuf0, sbuf1]
                for h in range(H2):
                    def ablk(b, acc2, h=h):
                        for ci in range(16):
                            colv = jnp.full((16,), ci, i32) + (b * 16 + h * DH)
                            xlc = plsc.load_gather(xlb, [rows, colv])
                            xrc = plsc.load_gather(xrb, [rows, colv + xr_col0])
                            u = xlc + xrc
                            lr = jnp.maximum(u, 0.2 * u)
                            attc = plsc.load_gather(att_v, [iota16, colv])
                            acc2 = acc2 + lr * attc
                        return acc2
                    a = lax.fori_loop(0, DH // 16, ablk, jnp.zeros((16,), f32))
                    sv = svs[h][pl.ds(tc * K + g * 16, 16)]
                    zs.append(jnp.exp(jnp.minimum(a - sv, 60.0)))

                def sblk(b, c3):
                    zz = jnp.where(b * 16 >= DH, zs[-1], zs[0])
                    for ci in range(16):
                        colv = jnp.full((16,), ci, i32) + b * 16
                        xlc = plsc.load_gather(xlb, [rows, colv])
                        plsc.store_scatter(xlb, [rows, colv], xlc * zz)
                    return c3

                lax.fori_loop(0, Deff // 16, sblk, 0)
                for j in range(16):
                    m = iota16 == j
                    plsc.addupdate_scatter(den0, [dv], zs[0], mask=m)
                    if H2 == 2:
                        plsc.addupdate_scatter(den1, [dv], zs[1], mask=m)
                return c2

            lax.fori_loop(0, K // 16, grp, 0)
            pltpu.sync_copy(xlb, acc.at[dstb], add=True)
            return c1

        lax.fori_loop(0, SUP // K, chunk_body, 0)
        return carry

    lax.fori_loop(0, ept // SUP, sup_body, 0)
    plsc.subcore_barrier()
    woff = core * N_PAD + sub * SP
    pltpu.sync_copy(acc.at[pl.ds(sub * SP, SP)], out_data.at[pl.ds(woff, SP)])
    wid = core * 16 + sub
    pltpu.sync_copy(den0, out_den0.at[wid])
    if H2 == 2:
        pltpu.sync_copy(den1, out_den1.at[wid])


def _sc_edge(xltab, xrtab, s, attf, src, dst, zeros, zerosd, Deff, H2, xr_col0,
             ept, split_edges, per_core_tables):
    mesh = plsc.VectorSubcoreMesh(core_axis_name="c", subcore_axis_name="s",
                                  num_cores=2, num_subcores=16)
    body = functools.partial(_sc_edge_body, Deff, H2, xr_col0, ept,
                             split_edges, per_core_tables)
    kern = pl.kernel(
        body,
        out_type=[
            jax.ShapeDtypeStruct((2 * N_PAD, TW), f32),
            jax.ShapeDtypeStruct((32, N), f32),
            jax.ShapeDtypeStruct((32, N), f32),
        ],
        mesh=mesh,
        compiler_params=pltpu.CompilerParams(needs_layout_passes=False),
        scratch_types=[
            pltpu.VMEM((16, TW), f32),
            pltpu.VMEM((K,), i32),
            pltpu.VMEM((K,), i32),
            pltpu.VMEM((SUP,), i32),
            pltpu.VMEM((SUP,), i32),
            pltpu.VMEM((SUP,), i32),
            pltpu.VMEM((SUP,) if H2 == 2 else (16,), i32),
            pltpu.VMEM((SUP,), f32),
            pltpu.VMEM((SUP,) if H2 == 2 else (16,), f32),
            pltpu.VMEM((K, TW), f32),
            pltpu.VMEM((K, TW), f32),
            pltpu.VMEM((N,), f32),
            pltpu.VMEM((N,) if H2 == 2 else (16,), f32),
            pltpu.VMEM((K,), i32),
            pltpu.VMEM_SHARED((N_PAD, TW), f32),
        ],
    )
    return kern(xltab, xrtab, s, attf, src, dst, zeros, zerosd)


def kernel(x, edge_index, batch, Wl1, Wr1, att1, b1, Wl2, Wr2, att2, b2,
           Wl3, Wr3, att3, b3, W_lin1, b_lin1, W_lin2, b_lin2):
    src = edge_index[0].astype(i32)
    dst = edge_index[1].astype(i32)
    bat3 = batch.astype(i32).reshape(GRID, 1, R)

    att1f = att1.reshape(1, 8 * DH)
    att2f = att2.reshape(1, DH)
    att3f = att3.reshape(1, DH)
    sel1 = (jnp.arange(8 * DH)[:, None] // DH
            == jnp.arange(8)[None, :]).astype(f32)
    zeros_tw = jnp.zeros((N_PAD, TW), f32)
    zerosd = jnp.zeros((N,), f32)

    xl1, xr1, s1 = _pre1(x, Wl1, Wr1, att1f, sel1)

    accs, den_pairs = [], []
    for c in range(2):
        lo = 256 * c
        xl_pair = jnp.concatenate([xl1[:, lo:lo + 128], xl1[:, lo + 128:lo + 256]],
                                  axis=0)
        xr_pair = jnp.concatenate([xr1[:, lo:lo + 128], xr1[:, lo + 128:lo + 256]],
                                  axis=0)
        s_pair = jnp.concatenate([s1[:, 4 * c:4 * c + 2], s1[:, 4 * c + 2:4 * c + 4]],
                                 axis=0)
        att_pair = jnp.concatenate([jnp.tile(att1f[:, lo:lo + 128], (16, 1)),
                                    jnp.tile(att1f[:, lo + 128:lo + 256], (16, 1))],
                                   axis=0)
        data, dn0, dn1 = _sc_edge(xl_pair, xr_pair, s_pair.reshape(-1), att_pair, src, dst,
                                  zeros_tw, zerosd, Deff=128, H2=2, xr_col0=0,
                                  ept=E // 16, split_edges=False,
                                  per_core_tables=True)
        accs.append(data[0:N])
        accs.append(data[N_PAD:N_PAD + N])
        den_pairs.append((dn0[0:16].reshape(16, GRID, R).transpose(1, 0, 2),
                          dn1[0:16].reshape(16, GRID, R).transpose(1, 0, 2)))
        den_pairs.append((dn0[16:32].reshape(16, GRID, R).transpose(1, 0, 2),
                          dn1[16:32].reshape(16, GRID, R).transpose(1, 0, 2)))

    p1, xlr2, s2 = _fin1pre2(accs, den_pairs, xl1, b1.reshape(1, -1), bat3,
                             Wl2, Wr2, att2f)

    data2, dn2, _ = _sc_edge(jnp.concatenate([xlr2[:, 0:DH], jnp.zeros((N, DH), f32)], axis=1), xlr2, s2.reshape(-1), jnp.tile(jnp.pad(att2f, ((0, 0), (0, TW - DH))), (32, 1)), src, dst,
                             zeros_tw, zerosd, Deff=DH, H2=1, xr_col0=DH,
                             ept=E // 32, split_edges=True,
                             per_core_tables=False)
    p2, xlr3, s3 = _finpre(data2[0:N], data2[N_PAD:N_PAD + N], dn2.reshape(32, GRID, R).transpose(1, 0, 2), xlr2,
                           b2.reshape(1, -1), bat3, Wl3, Wr3, att3f)

    data3, dn3, _ = _sc_edge(jnp.concatenate([xlr3[:, 0:DH], jnp.zeros((N, DH), f32)], axis=1), xlr3, s3.reshape(-1), jnp.tile(jnp.pad(att3f, ((0, 0), (0, TW - DH))), (32, 1)), src, dst,
                             zeros_tw, zerosd, Deff=DH, H2=1, xr_col0=DH,
                             ept=E // 32, split_edges=True,
                             per_core_tables=False)
    p3, xlr4, s4 = _finpre(data3[0:N], data3[N_PAD:N_PAD + N], dn3.reshape(32, GRID, R).transpose(1, 0, 2), xlr3,
                           b3.reshape(1, -1), bat3, Wl3, Wr3, att3f)

    data4, dn4, _ = _sc_edge(jnp.concatenate([xlr4[:, 0:DH], jnp.zeros((N, DH), f32)], axis=1), xlr4, s4.reshape(-1), jnp.tile(jnp.pad(att3f, ((0, 0), (0, TW - DH))), (32, 1)), src, dst,
                             zeros_tw, zerosd, Deff=DH, H2=1, xr_col0=DH,
                             ept=E // 32, split_edges=True,
                             per_core_tables=False)
    p4 = _fin4(data4[0:N], data4[N_PAD:N_PAD + N], dn4.reshape(32, GRID, R).transpose(1, 0, 2), xlr4,
               b3.reshape(1, -1), bat3)[0]

    h_cat = jnp.concatenate([p1, p2, p3, p4], axis=1)
    return _mlp_tail(h_cat, W_lin1, b_lin1, W_lin2, b_lin2)
